# baseline (speedup 1.0000x reference)
import numpy as np

import jax
import jax.numpy as jnp
from jax import lax
from jax.experimental import pallas as pl
from jax.experimental.pallas import tpu as pltpu
from jax.experimental.pallas import tpu_sc as plsc

N = 8192
V = 8192
K = 2048
TN = 256
GRID = N // TN
L = 16
NVEC = N // L

_INTERPRET = False

_SIGN = int(np.int32(np.uint32(0x80000000)))


def _stats_body(logits_ref, keys_ref, lse_ref):
    x = logits_ref[...]
    m = jnp.max(x, axis=1, keepdims=True)
    xm = x - m
    e = jnp.exp(xm)
    s = jnp.sum(e, axis=1, keepdims=True)
    w = jnp.sum(e * xm, axis=1, keepdims=True)
    logs = jnp.log(s)
    ent = logs - w / s
    b = lax.bitcast_convert_type(ent, jnp.int32)
    keys_ref[...] = jnp.where(b < 0, b ^ 0x7FFFFFFF, b)
    lse_ref[...] = m + logs


def _stage1(logits_flat):
    return pl.pallas_call(
        _stats_body,
        grid=(GRID,),
        in_specs=[
            pl.BlockSpec((TN, V), lambda i: (i, 0)),
        ],
        out_specs=[
            pl.BlockSpec((TN, 1), lambda i: (i, 0)),
            pl.BlockSpec((TN, 1), lambda i: (i, 0)),
        ],
        out_shape=[
            jax.ShapeDtypeStruct((N, 1), jnp.int32),
            jax.ShapeDtypeStruct((N, 1), jnp.float32),
        ],
        interpret=_INTERPRET,
    )(logits_flat)


def _select_body(keys_hbm, lse_hbm, tgt_hbm, logits_hbm, out_hbm,
                 keys_v, lse_v, tgt_v, idx_v, xtl_v, xt_v, hist_v, out_v,
                 xt_sh, sem):
    cid = lax.axis_index("c")
    sid = lax.axis_index("s")
    EPT = N // 16

    @pl.when(cid == 0)
    def _():
        lane = lax.iota(jnp.int32, L)
        ones16 = jnp.full((L,), 1, jnp.int32)
        shiftv = lambda n: jnp.full((L,), n, jnp.int32)

        base = sid * EPT
        pltpu.sync_copy(tgt_hbm.at[pl.ds(base, EPT)], tgt_v)

        def idx_body(j, carry):
            for u in range(4):
                i = j * 4 + u
                t16 = tgt_v[pl.ds(i * L, L)]
                flat = (base + i * L + lane) * V + t16
                idx_v[i // 8, pl.ds((i % 8) * L, L)] = flat
            return carry

        lax.fori_loop(0, EPT // L // 4, idx_body, 0)

        def fire_body(jj, carry):
            pltpu.async_copy(logits_hbm.at[idx_v.at[jj]],
                             xtl_v.at[pl.ds(jj * 128, 128)], sem)
            return carry

        lax.fori_loop(0, 4, fire_body, 0)

        def drain_stage():
            def drain_body(jj, carry):
                pltpu.make_async_copy(
                    logits_hbm.at[idx_v.at[0]],
                    xtl_v.at[pl.ds(0, 128)], sem).wait()
                return carry

            lax.fori_loop(0, 4, drain_body, 0)
            pltpu.sync_copy(xtl_v, xt_sh.at[sid])

        @pl.when(sid != 0)
        def _other_tiles():
            drain_stage()
            plsc.subcore_barrier()

    @pl.when((cid == 0) & (sid == 0))
    def _():
        lane = lax.iota(jnp.int32, L)
        ones16 = jnp.full((L,), 1, jnp.int32)
        shiftv = lambda n: jnp.full((L,), n, jnp.int32)

        def drain_stage():
            def drain_body(jj, carry):
                pltpu.make_async_copy(
                    logits_hbm.at[idx_v.at[0]],
                    xtl_v.at[pl.ds(0, 128)], sem).wait()
                return carry

            lax.fori_loop(0, 4, drain_body, 0)
            pltpu.sync_copy(xtl_v, xt_sh.at[sid])

        pltpu.sync_copy(keys_hbm, keys_v)
        pltpu.sync_copy(lse_hbm, lse_v)

        prefix = jnp.int32(0)
        r = jnp.int32(K)
        for p in range(4):
            shift = 24 - 8 * p
            hmask = int(np.int32(np.uint32((0xFFFFFFFF << (shift + 8))
                                           & 0xFFFFFFFF))) if p else 0

            def zero_body(j, carry):
                for u in range(8):
                    hist_v[pl.ds((j * 8 + u) * L, L)] = jnp.zeros(
                        (L,), jnp.int32)
                return carry

            lax.fori_loop(0, 256 // 8, zero_body, 0)

            pref_c = prefix

            def hist_body(j, carry):
                for u in range(8):
                    i = j * 8 + u
                    kv = keys_v[pl.ds(i * L, L)]
                    ub = kv ^ _SIGN
                    digit = lax.shift_right_logical(ub, shiftv(shift)) & 255
                    if p == 0:
                        plsc.addupdate_scatter(
                            hist_v, [digit * L + lane], ones16)
                    else:
                        mch = (ub & hmask) == pref_c
                        plsc.addupdate_scatter(
                            hist_v, [digit * L + lane], ones16, mask=mch)
                return carry

            lax.fori_loop(0, NVEC // 8, hist_body, 0)

            def scan_body(jb, carry):
                c, dstar, found = carry
                dbase = (15 - jb) * L
                tots = [jnp.sum(hist_v[pl.ds((dbase + 15 - u) * L, L)])
                        for u in range(L)]
                for u in range(L):
                    h = tots[u]
                    d = dbase + 15 - u
                    done = jnp.logical_and(found == 0, c + h >= r)
                    dstar = jnp.where(done, d, dstar)
                    found = jnp.where(done, jnp.int32(1), found)
                    c = jnp.where(found == 1, c, c + h)
                return (c, dstar, found)

            c, dstar, _ = lax.fori_loop(
                0, 16, scan_body,
                (jnp.int32(0), jnp.int32(0), jnp.int32(0)))
            r = r - c
            prefix = prefix | lax.shift_left(dstar, shift)

        t_s = prefix ^ _SIGN

        drain_stage()
        plsc.subcore_barrier()
        pltpu.sync_copy(xt_sh, xt_v)

        zf = jnp.zeros((L,), jnp.float32)
        zi = jnp.zeros((L,), jnp.int32)

        def fin_body(j, carry):
            agt, aeq, ceq = carry
            for u in range(8):
                i = j * 8 + u
                kv = keys_v[pl.ds(i * L, L)]
                nv = (lse_v[pl.ds(i * L, L)]
                      - xt_v[i // 32, pl.ds((i % 32) * L, L)])
                gt = kv > t_s
                eq = kv == t_s
                agt = agt + jnp.where(gt, nv, zf)
                aeq = aeq + jnp.where(eq, nv, zf)
                ceq = ceq + jnp.where(eq, ones16, zi)
            return (agt, aeq, ceq)

        agt, aeq, ceq = lax.fori_loop(0, NVEC // 8, fin_body, (zf, zf, zi))
        acc_gt = jnp.sum(agt)
        acc_eq = jnp.sum(aeq)
        cnt_eq = jnp.sum(ceq)

        def all_ties(_):
            return acc_eq

        def ranked_ties(_):
            def tie_body(i, carry):
                acc, tcnt = carry
                kv = keys_v[pl.ds(i * L, L)]
                nv = (lse_v[pl.ds(i * L, L)]
                      - xt_v[i // 32, pl.ds((i % 32) * L, L)])
                eqc = (kv == t_s).astype(jnp.int32)
                incl = jnp.cumsum(eqc) + tcnt
                take = jnp.logical_and(eqc == 1, incl <= r)
                acc = acc + jnp.sum(jnp.where(take, nv, 0.0))
                tcnt = tcnt + jnp.sum(eqc)
                return (acc, tcnt)

            acc, _ = lax.fori_loop(0, NVEC, tie_body,
                                   (jnp.float32(0), jnp.int32(0)))
            return acc

        acc_tie = lax.cond(cnt_eq == r, all_ties, ranked_ties, 0)
        acc = acc_gt + acc_tie
        out_v[...] = jnp.full((L,), acc * (1.0 / K), jnp.float32)
        pltpu.sync_copy(out_v, out_hbm)


_select = pl.kernel(
    _select_body,
    out_type=jax.ShapeDtypeStruct((L,), jnp.float32),
    mesh=plsc.VectorSubcoreMesh(core_axis_name="c", subcore_axis_name="s"),
    compiler_params=pltpu.CompilerParams(needs_layout_passes=False),
    scratch_types=[
        pltpu.VMEM((N,), jnp.int32),
        pltpu.VMEM((N,), jnp.float32),
        pltpu.VMEM((512,), jnp.int32),
        pltpu.VMEM((4, 128), jnp.int32),
        pltpu.VMEM((512,), jnp.float32),
        pltpu.VMEM((16, 512), jnp.float32),
        pltpu.VMEM((256 * L,), jnp.int32),
        pltpu.VMEM((L,), jnp.float32),
        pltpu.VMEM_SHARED((16, 512), jnp.float32),
        pltpu.SemaphoreType.DMA,
    ],
)


def kernel(logits, targets):
    logits_flat = logits.reshape(N, V)
    keys, lse = _stage1(logits_flat)
    sel = _select(keys.reshape(N), lse.reshape(N), targets.reshape(N),
                  logits.reshape(N * V))
    return sel[0]


# speedup vs baseline: 1.9542x; 1.9542x over previous
import numpy as np

import jax
import jax.numpy as jnp
from jax import lax
from jax.experimental import pallas as pl
from jax.experimental.pallas import tpu as pltpu
from jax.experimental.pallas import tpu_sc as plsc

N = 8192
V = 8192
K = 2048
TN = 256
GRID = N // TN
L = 16
NVEC = N // L


_SIGN = int(np.int32(np.uint32(0x80000000)))


def _stats_body(targets_ref, logits_ref, keys_ref, nll_ref):
    x = logits_ref[...]
    t = targets_ref[...]
    col = lax.broadcasted_iota(jnp.int32, (TN, V), 1)
    m = jnp.max(x, axis=1, keepdims=True)
    xt = jnp.sum(jnp.where(col == t, x, 0.0), axis=1, keepdims=True)
    xm = x - m
    e = jnp.exp(xm)
    s = jnp.sum(e, axis=1, keepdims=True)
    w = jnp.sum(e * xm, axis=1, keepdims=True)
    logs = jnp.log(s)
    ent = logs - w / s
    nll = (m + logs) - xt
    b = lax.bitcast_convert_type(ent, jnp.int32)
    keys_ref[...] = jnp.where(b < 0, b ^ 0x7FFFFFFF, b)
    nll_ref[...] = nll


def _stage1(logits_flat, targets_col):
    return pl.pallas_call(
        _stats_body,
        grid=(GRID,),
        in_specs=[
            pl.BlockSpec((TN, 1), lambda i: (i, 0)),
            pl.BlockSpec((TN, V), lambda i: (i, 0)),
        ],
        out_specs=[
            pl.BlockSpec((TN, 1), lambda i: (i, 0)),
            pl.BlockSpec((TN, 1), lambda i: (i, 0)),
        ],
        out_shape=[
            jax.ShapeDtypeStruct((N, 1), jnp.int32),
            jax.ShapeDtypeStruct((N, 1), jnp.float32),
        ],
    )(targets_col, logits_flat)


def _select_body(keys_hbm, nll_hbm, out_hbm, keys_v, nll_v, hist_v, out_v):
    @pl.when((lax.axis_index("c") == 0) & (lax.axis_index("s") == 0))
    def _():
        pltpu.sync_copy(keys_hbm, keys_v)
        pltpu.sync_copy(nll_hbm, nll_v)
        lane = lax.iota(jnp.int32, L)
        ones16 = jnp.full((L,), 1, jnp.int32)
        shiftv = lambda n: jnp.full((L,), n, jnp.int32)

        prefix = jnp.int32(0)
        r = jnp.int32(K)
        for p in range(4):
            shift = 24 - 8 * p
            hmask = int(np.int32(np.uint32((0xFFFFFFFF << (shift + 8))
                                           & 0xFFFFFFFF))) if p else 0

            def zero_body(j, carry):
                for u in range(8):
                    hist_v[pl.ds((j * 8 + u) * L, L)] = jnp.zeros(
                        (L,), jnp.int32)
                return carry

            lax.fori_loop(0, 256 // 8, zero_body, 0)

            pref_c = prefix

            def hist_body(j, carry):
                for u in range(8):
                    i = j * 8 + u
                    kv = keys_v[pl.ds(i * L, L)]
                    ub = kv ^ _SIGN
                    digit = lax.shift_right_logical(ub, shiftv(shift)) & 255
                    if p == 0:
                        plsc.addupdate_scatter(
                            hist_v, [digit * L + lane], ones16)
                    else:
                        mch = (ub & hmask) == pref_c
                        plsc.addupdate_scatter(
                            hist_v, [digit * L + lane], ones16, mask=mch)
                return carry

            lax.fori_loop(0, NVEC // 8, hist_body, 0)

            def scan_body(jb, carry):
                c, dstar, found = carry
                dbase = (15 - jb) * L
                tots = [jnp.sum(hist_v[pl.ds((dbase + 15 - u) * L, L)])
                        for u in range(L)]
                for u in range(L):
                    h = tots[u]
                    d = dbase + 15 - u
                    done = jnp.logical_and(found == 0, c + h >= r)
                    dstar = jnp.where(done, d, dstar)
                    found = jnp.where(done, jnp.int32(1), found)
                    c = jnp.where(found == 1, c, c + h)
                return (c, dstar, found)

            c, dstar, _ = lax.fori_loop(
                0, 16, scan_body,
                (jnp.int32(0), jnp.int32(0), jnp.int32(0)))
            r = r - c
            prefix = prefix | lax.shift_left(dstar, shift)

        t_s = prefix ^ _SIGN

        zf = jnp.zeros((L,), jnp.float32)
        zi = jnp.zeros((L,), jnp.int32)

        def fin_body(j, carry):
            agt, aeq, ceq = carry
            for u in range(8):
                i = j * 8 + u
                kv = keys_v[pl.ds(i * L, L)]
                nv = nll_v[pl.ds(i * L, L)]
                gt = kv > t_s
                eq = kv == t_s
                agt = agt + jnp.where(gt, nv, zf)
                aeq = aeq + jnp.where(eq, nv, zf)
                ceq = ceq + jnp.where(eq, ones16, zi)
            return (agt, aeq, ceq)

        agt, aeq, ceq = lax.fori_loop(0, NVEC // 8, fin_body, (zf, zf, zi))
        acc_gt = jnp.sum(agt)
        acc_eq = jnp.sum(aeq)
        cnt_eq = jnp.sum(ceq)

        def all_ties(_):
            return acc_eq

        def ranked_ties(_):
            def tie_body(i, carry):
                acc, tcnt = carry
                kv = keys_v[pl.ds(i * L, L)]
                nv = nll_v[pl.ds(i * L, L)]
                eqc = (kv == t_s).astype(jnp.int32)
                incl = jnp.cumsum(eqc) + tcnt
                take = jnp.logical_and(eqc == 1, incl <= r)
                acc = acc + jnp.sum(jnp.where(take, nv, 0.0))
                tcnt = tcnt + jnp.sum(eqc)
                return (acc, tcnt)

            acc, _ = lax.fori_loop(0, NVEC, tie_body,
                                   (jnp.float32(0), jnp.int32(0)))
            return acc

        acc_tie = lax.cond(cnt_eq == r, all_ties, ranked_ties, 0)
        acc = acc_gt + acc_tie
        out_v[...] = jnp.full((L,), acc * (1.0 / K), jnp.float32)
        pltpu.sync_copy(out_v, out_hbm)


_select = pl.kernel(
    _select_body,
    out_type=jax.ShapeDtypeStruct((L,), jnp.float32),
    mesh=plsc.VectorSubcoreMesh(core_axis_name="c", subcore_axis_name="s"),
    compiler_params=pltpu.CompilerParams(needs_layout_passes=False),
    scratch_types=[
        pltpu.VMEM((N,), jnp.int32),
        pltpu.VMEM((N,), jnp.float32),
        pltpu.VMEM((256 * L,), jnp.int32),
        pltpu.VMEM((L,), jnp.float32),
    ],
)


def kernel(logits, targets):
    logits_flat = logits.reshape(N, V)
    targets_col = targets.reshape(N, 1)
    keys, nll = _stage1(logits_flat, targets_col)
    sel = _select(keys.reshape(N), nll.reshape(N))
    return sel[0]


# speedup vs baseline: 1.9667x; 1.0064x over previous
import numpy as np

import jax
import jax.numpy as jnp
from jax import lax
from jax.experimental import pallas as pl
from jax.experimental.pallas import tpu as pltpu
from jax.experimental.pallas import tpu_sc as plsc

N = 8192
V = 8192
K = 2048
TN = 256
GRID = N // TN
L = 16
NVEC = N // L


_SIGN = int(np.int32(np.uint32(0x80000000)))


def _stats_body(targets_ref, logits_ref, keys_ref, nll_ref):
    x = logits_ref[...]
    t = targets_ref[...]
    col = lax.broadcasted_iota(jnp.int32, (TN, V), 1)
    m = jnp.max(x, axis=1, keepdims=True)
    xt = jnp.sum(jnp.where(col == t, x, 0.0), axis=1, keepdims=True)
    xm = x - m
    e = jnp.exp(xm)
    s = jnp.sum(e, axis=1, keepdims=True)
    w = jnp.sum(e * xm, axis=1, keepdims=True)
    logs = jnp.log(s)
    ent = logs - w / s
    nll = (m + logs) - xt
    b = lax.bitcast_convert_type(ent, jnp.int32)
    keys_ref[...] = jnp.where(b < 0, b ^ 0x7FFFFFFF, b)
    nll_ref[...] = nll


def _stage1(logits_flat, targets_col):
    return pl.pallas_call(
        _stats_body,
        grid=(GRID,),
        in_specs=[
            pl.BlockSpec((TN, 1), lambda i: (i, 0)),
            pl.BlockSpec((TN, V), lambda i: (i, 0)),
        ],
        out_specs=[
            pl.BlockSpec((TN, 1), lambda i: (i, 0)),
            pl.BlockSpec((TN, 1), lambda i: (i, 0)),
        ],
        out_shape=[
            jax.ShapeDtypeStruct((N, 1), jnp.int32),
            jax.ShapeDtypeStruct((N, 1), jnp.float32),
        ],
    )(targets_col, logits_flat)


def _select_body(keys_hbm, nll_hbm, out_hbm, keys_v, nll_v, hist_v, out_v,
                 sem):
    @pl.when((lax.axis_index("c") == 0) & (lax.axis_index("s") == 0))
    def _():
        cp_k = pltpu.make_async_copy(keys_hbm, keys_v, sem)
        cp_n = pltpu.make_async_copy(nll_hbm, nll_v, sem)
        cp_k.start()
        cp_n.start()
        cp_k.wait()
        cp_n.wait()
        lane = lax.iota(jnp.int32, L)
        ones16 = jnp.full((L,), 1, jnp.int32)
        shiftv = lambda n: jnp.full((L,), n, jnp.int32)

        prefix = jnp.int32(0)
        r = jnp.int32(K)
        for p in range(4):
            shift = 24 - 8 * p
            hmask = int(np.int32(np.uint32((0xFFFFFFFF << (shift + 8))
                                           & 0xFFFFFFFF))) if p else 0

            def zero_body(j, carry):
                for u in range(8):
                    hist_v[pl.ds((j * 8 + u) * L, L)] = jnp.zeros(
                        (L,), jnp.int32)
                return carry

            lax.fori_loop(0, 256 // 8, zero_body, 0)

            pref_c = prefix

            def hist_body(j, carry):
                for u in range(8):
                    i = j * 8 + u
                    kv = keys_v[pl.ds(i * L, L)]
                    ub = kv ^ _SIGN
                    digit = lax.shift_right_logical(ub, shiftv(shift)) & 255
                    if p == 0:
                        plsc.addupdate_scatter(
                            hist_v, [digit * L + lane], ones16)
                    else:
                        mch = (ub & hmask) == pref_c
                        plsc.addupdate_scatter(
                            hist_v, [digit * L + lane], ones16, mask=mch)
                return carry

            lax.fori_loop(0, NVEC // 8, hist_body, 0)

            def scan_body(jb, carry):
                c, dstar, found = carry
                dbase = (15 - jb) * L
                tots = [jnp.sum(hist_v[pl.ds((dbase + 15 - u) * L, L)])
                        for u in range(L)]
                for u in range(L):
                    h = tots[u]
                    d = dbase + 15 - u
                    done = jnp.logical_and(found == 0, c + h >= r)
                    dstar = jnp.where(done, d, dstar)
                    found = jnp.where(done, jnp.int32(1), found)
                    c = jnp.where(found == 1, c, c + h)
                return (c, dstar, found)

            def scan_cond(carry):
                jb, (c, dstar, found) = carry
                return jnp.logical_and(jb < 16, found == 0)

            def scan_step(carry):
                jb, st = carry
                return (jb + 1, scan_body(jb, st))

            _, (c, dstar, _) = lax.while_loop(
                scan_cond, scan_step,
                (jnp.int32(0),
                 (jnp.int32(0), jnp.int32(0), jnp.int32(0))))
            r = r - c
            prefix = prefix | lax.shift_left(dstar, shift)

        t_s = prefix ^ _SIGN

        zf = jnp.zeros((L,), jnp.float32)
        zi = jnp.zeros((L,), jnp.int32)

        def fin_body(j, carry):
            agt, aeq, ceq = carry
            for u in range(8):
                i = j * 8 + u
                kv = keys_v[pl.ds(i * L, L)]
                nv = nll_v[pl.ds(i * L, L)]
                gt = kv > t_s
                eq = kv == t_s
                agt = agt + jnp.where(gt, nv, zf)
                aeq = aeq + jnp.where(eq, nv, zf)
                ceq = ceq + jnp.where(eq, ones16, zi)
            return (agt, aeq, ceq)

        agt, aeq, ceq = lax.fori_loop(0, NVEC // 8, fin_body, (zf, zf, zi))
        acc_gt = jnp.sum(agt)
        acc_eq = jnp.sum(aeq)
        cnt_eq = jnp.sum(ceq)

        def all_ties(_):
            return acc_eq

        def ranked_ties(_):
            def tie_body(i, carry):
                acc, tcnt = carry
                kv = keys_v[pl.ds(i * L, L)]
                nv = nll_v[pl.ds(i * L, L)]
                eqc = (kv == t_s).astype(jnp.int32)
                incl = jnp.cumsum(eqc) + tcnt
                take = jnp.logical_and(eqc == 1, incl <= r)
                acc = acc + jnp.sum(jnp.where(take, nv, 0.0))
                tcnt = tcnt + jnp.sum(eqc)
                return (acc, tcnt)

            acc, _ = lax.fori_loop(0, NVEC, tie_body,
                                   (jnp.float32(0), jnp.int32(0)))
            return acc

        acc_tie = lax.cond(cnt_eq == r, all_ties, ranked_ties, 0)
        acc = acc_gt + acc_tie
        out_v[...] = jnp.full((L,), acc * (1.0 / K), jnp.float32)
        pltpu.sync_copy(out_v, out_hbm)


_select = pl.kernel(
    _select_body,
    out_type=jax.ShapeDtypeStruct((L,), jnp.float32),
    mesh=plsc.VectorSubcoreMesh(core_axis_name="c", subcore_axis_name="s"),
    compiler_params=pltpu.CompilerParams(needs_layout_passes=False),
    scratch_types=[
        pltpu.VMEM((N,), jnp.int32),
        pltpu.VMEM((N,), jnp.float32),
        pltpu.VMEM((256 * L,), jnp.int32),
        pltpu.VMEM((L,), jnp.float32),
        pltpu.SemaphoreType.DMA,
    ],
)


def kernel(logits, targets):
    logits_flat = logits.reshape(N, V)
    targets_col = targets.reshape(N, 1)
    keys, nll = _stage1(logits_flat, targets_col)
    sel = _select(keys.reshape(N), nll.reshape(N))
    return sel[0]
